# baseline (speedup 1.0000x reference)
import functools

import jax
import jax.numpy as jnp
from jax.experimental import pallas as pl
from jax.experimental.pallas import tpu as pltpu
from jax.experimental.pallas import tpu_sc as plsc

PAD_IDX = 0
HEAD_DIM = 64
ROPE_THETA = 10000.0

_NUM_CORES = 2
_NUM_SUBCORES = 16
_CHUNK = 16
_NBUF = 5
_AHEAD = 2


def _sc_gather(table, idx_flat, b, s, hidden):
    n_tokens = b * s
    n_workers = _NUM_CORES * _NUM_SUBCORES
    per_w = n_tokens // n_workers
    n_chunks = per_w // _CHUNK
    w_per_batch = s // per_w

    @functools.partial(
        pl.kernel,
        out_type=jax.ShapeDtypeStruct((b, s, hidden), table.dtype),
        mesh=plsc.VectorSubcoreMesh(core_axis_name="core",
                                    subcore_axis_name="subcore"),
        scratch_types=(
            [pltpu.VMEM((per_w,), jnp.int32)]
            + [pltpu.VMEM((_CHUNK, hidden), jnp.float32)] * _NBUF
            + [pltpu.SemaphoreType.DMA] * (2 * _NBUF)
        ),
    )
    def gather_kernel(x_hbm, i_hbm, o_hbm, idx_v, *bufs_and_sems):
        bufs = bufs_and_sems[:_NBUF]
        gsems = bufs_and_sems[_NBUF:2 * _NBUF]
        ssems = bufs_and_sems[2 * _NBUF:]
        wid = (jax.lax.axis_index("subcore") * _NUM_CORES
               + jax.lax.axis_index("core"))
        base = wid * per_w
        batch_i = wid // w_per_batch
        seq_0 = (wid % w_per_batch) * per_w
        pltpu.sync_copy(i_hbm.at[pl.ds(base, per_w)], idx_v)

        def gather_copy(c, n):
            return pltpu.make_async_copy(
                x_hbm.at[idx_v.at[pl.ds(c * _CHUNK, _CHUNK)]],
                bufs[n], gsems[n])

        def scatter_copy(c, n):
            return pltpu.make_async_copy(
                bufs[n],
                o_hbm.at[batch_i, pl.ds(seq_0 + c * _CHUNK, _CHUNK)],
                ssems[n])

        rounds = (n_chunks - _NBUF - _AHEAD) // _NBUF
        assert n_chunks == _NBUF + rounds * _NBUF + _AHEAD
        assert _AHEAD < _NBUF <= n_chunks

        def steady_step(c, b):
            b2 = (b + _AHEAD) % _NBUF
            scatter_copy(c + _AHEAD - _NBUF, b2).wait()
            gather_copy(c + _AHEAD, b2).start()
            gather_copy(c, b).wait()
            scatter_copy(c, b).start()

        for a in range(_AHEAD):
            gather_copy(a, a).start()
        for c in range(_NBUF):
            j = c + _AHEAD - _NBUF
            if j >= 0:
                scatter_copy(j, (c + _AHEAD) % _NBUF).wait()
            gather_copy(c + _AHEAD, (c + _AHEAD) % _NBUF).start()
            gather_copy(c, c).wait()
            scatter_copy(c, c).start()

        @pl.loop(0, rounds)
        def _(r):
            c0 = _NBUF + r * _NBUF
            for b in range(_NBUF):
                steady_step(c0 + b, b)

        for c in range(n_chunks - _AHEAD, n_chunks):
            b = c % _NBUF
            scatter_copy(c + _AHEAD - _NBUF, (b + _AHEAD) % _NBUF).wait()
            gather_copy(c, b).wait()
            scatter_copy(c, b).start()
        for c in range(n_chunks - (_NBUF - _AHEAD), n_chunks):
            scatter_copy(c, c % _NBUF).wait()

    return gather_kernel(table, idx_flat)


def _rope_body(ids_ref, invf_ref, pos_ref, mask_ref, cos_ref, sin_ref):
    ids = ids_ref[...]
    b, s = ids.shape
    pos_ref[...] = jax.lax.broadcasted_iota(jnp.int32, (b, s), 1)
    mask_ref[...] = (ids != PAD_IDX).astype(jnp.int32)
    pos3 = jax.lax.broadcasted_iota(jnp.int32, (b, HEAD_DIM, s), 2).astype(
        jnp.float32)
    phase = pos3 * invf_ref[...]
    cos_ref[...] = jnp.cos(phase)
    sin_ref[...] = jnp.sin(phase)


def _tc_rope(input_ids, invf_full):
    b, s = input_ids.shape
    return pl.pallas_call(
        _rope_body,
        out_shape=(
            jax.ShapeDtypeStruct((b, s), jnp.int32),
            jax.ShapeDtypeStruct((b, s), jnp.int32),
            jax.ShapeDtypeStruct((b, HEAD_DIM, s), jnp.float32),
            jax.ShapeDtypeStruct((b, HEAD_DIM, s), jnp.float32),
        ),
    )(input_ids, invf_full)


def kernel(input_ids, embed_table):
    b, s = input_ids.shape
    vocab, hidden = embed_table.shape
    n_tokens = b * s

    idx_flat = input_ids.reshape(n_tokens)
    hidden_states = _sc_gather(embed_table, idx_flat, b, s, hidden)

    inv_freq = 1.0 / (ROPE_THETA ** (
        jnp.arange(0, HEAD_DIM, 2, dtype=jnp.float32) / HEAD_DIM))
    invf_full = jnp.concatenate([inv_freq, inv_freq]).reshape(1, HEAD_DIM, 1)

    position_ids, attention_mask, cos_t, sin_t = _tc_rope(input_ids, invf_full)
    cos = jnp.swapaxes(cos_t, 1, 2)
    sin = jnp.swapaxes(sin_t, 1, 2)
    return (hidden_states, position_ids, attention_mask, cos, sin, input_ids)


# speedup vs baseline: 1.0070x; 1.0070x over previous
import functools

import jax
import jax.numpy as jnp
from jax.experimental import pallas as pl
from jax.experimental.pallas import tpu as pltpu
from jax.experimental.pallas import tpu_sc as plsc

PAD_IDX = 0
HEAD_DIM = 64
ROPE_THETA = 10000.0

_NUM_CORES = 2
_NUM_SUBCORES = 16
_CHUNK = 16
_NBUF = 6
_AHEAD = 2


def _sc_gather(table, idx_flat, b, s, hidden):
    n_tokens = b * s
    n_workers = _NUM_CORES * _NUM_SUBCORES
    per_w = n_tokens // n_workers
    n_chunks = per_w // _CHUNK
    w_per_batch = s // per_w

    @functools.partial(
        pl.kernel,
        out_type=jax.ShapeDtypeStruct((b, s, hidden), table.dtype),
        mesh=plsc.VectorSubcoreMesh(core_axis_name="core",
                                    subcore_axis_name="subcore"),
        scratch_types=(
            [pltpu.VMEM((per_w,), jnp.int32)]
            + [pltpu.VMEM((_CHUNK, hidden), jnp.float32)] * _NBUF
            + [pltpu.SemaphoreType.DMA] * (2 * _NBUF)
        ),
    )
    def gather_kernel(x_hbm, i_hbm, o_hbm, idx_v, *bufs_and_sems):
        bufs = bufs_and_sems[:_NBUF]
        gsems = bufs_and_sems[_NBUF:2 * _NBUF]
        ssems = bufs_and_sems[2 * _NBUF:]
        wid = (jax.lax.axis_index("subcore") * _NUM_CORES
               + jax.lax.axis_index("core"))
        base = wid * per_w
        batch_i = wid // w_per_batch
        seq_0 = (wid % w_per_batch) * per_w
        def gather_copy(c, n):
            return pltpu.make_async_copy(
                x_hbm.at[idx_v.at[pl.ds(c * _CHUNK, _CHUNK)]],
                bufs[n], gsems[n])

        def scatter_copy(c, n):
            return pltpu.make_async_copy(
                bufs[n],
                o_hbm.at[batch_i, pl.ds(seq_0 + c * _CHUNK, _CHUNK)],
                ssems[n])

        rounds = (n_chunks - _NBUF - _AHEAD) // _NBUF
        assert n_chunks == _NBUF + rounds * _NBUF + _AHEAD
        assert _AHEAD < _NBUF <= n_chunks

        def steady_step(c, b):
            b2 = (b + _AHEAD) % _NBUF
            scatter_copy(c + _AHEAD - _NBUF, b2).wait()
            gather_copy(c + _AHEAD, b2).start()
            gather_copy(c, b).wait()
            scatter_copy(c, b).start()

        head_idx = _AHEAD * _CHUNK
        pltpu.sync_copy(i_hbm.at[pl.ds(base, head_idx)],
                        idx_v.at[pl.ds(0, head_idx)])
        for a in range(_AHEAD):
            gather_copy(a, a).start()
        pltpu.sync_copy(i_hbm.at[pl.ds(base + head_idx, per_w - head_idx)],
                        idx_v.at[pl.ds(head_idx, per_w - head_idx)])
        for c in range(_NBUF):
            j = c + _AHEAD - _NBUF
            if j >= 0:
                scatter_copy(j, (c + _AHEAD) % _NBUF).wait()
            gather_copy(c + _AHEAD, (c + _AHEAD) % _NBUF).start()
            gather_copy(c, c).wait()
            scatter_copy(c, c).start()

        @pl.loop(0, rounds)
        def _(r):
            c0 = _NBUF + r * _NBUF
            for b in range(_NBUF):
                steady_step(c0 + b, b)

        for c in range(n_chunks - _AHEAD, n_chunks):
            b = c % _NBUF
            scatter_copy(c + _AHEAD - _NBUF, (b + _AHEAD) % _NBUF).wait()
            gather_copy(c, b).wait()
            scatter_copy(c, b).start()
        for c in range(n_chunks - (_NBUF - _AHEAD), n_chunks):
            scatter_copy(c, c % _NBUF).wait()

    return gather_kernel(table, idx_flat)


def _rope_body(ids_ref, invf_ref, pos_ref, mask_ref, cos_ref, sin_ref):
    ids = ids_ref[...]
    b, s = ids.shape
    pos_ref[...] = jax.lax.broadcasted_iota(jnp.int32, (b, s), 1)
    mask_ref[...] = (ids != PAD_IDX).astype(jnp.int32)
    pos3 = jax.lax.broadcasted_iota(jnp.int32, (b, HEAD_DIM, s), 2).astype(
        jnp.float32)
    phase = pos3 * invf_ref[...]
    cos_ref[...] = jnp.cos(phase)
    sin_ref[...] = jnp.sin(phase)


def _tc_rope(input_ids, invf_full):
    b, s = input_ids.shape
    return pl.pallas_call(
        _rope_body,
        out_shape=(
            jax.ShapeDtypeStruct((b, s), jnp.int32),
            jax.ShapeDtypeStruct((b, s), jnp.int32),
            jax.ShapeDtypeStruct((b, HEAD_DIM, s), jnp.float32),
            jax.ShapeDtypeStruct((b, HEAD_DIM, s), jnp.float32),
        ),
    )(input_ids, invf_full)


def kernel(input_ids, embed_table):
    b, s = input_ids.shape
    vocab, hidden = embed_table.shape
    n_tokens = b * s

    idx_flat = input_ids.reshape(n_tokens)
    hidden_states = _sc_gather(embed_table, idx_flat, b, s, hidden)

    inv_freq = 1.0 / (ROPE_THETA ** (
        jnp.arange(0, HEAD_DIM, 2, dtype=jnp.float32) / HEAD_DIM))
    invf_full = jnp.concatenate([inv_freq, inv_freq]).reshape(1, HEAD_DIM, 1)

    position_ids, attention_mask, cos_t, sin_t = _tc_rope(input_ids, invf_full)
    cos = jnp.swapaxes(cos_t, 1, 2)
    sin = jnp.swapaxes(sin_t, 1, 2)
    return (hidden_states, position_ids, attention_mask, cos, sin, input_ids)
